# baseline (speedup 1.0000x reference)
import functools
import math

import jax
import jax.numpy as jnp
from jax import lax
from jax.experimental import pallas as pl
from jax.experimental.pallas import tpu as pltpu
import jax.experimental.pallas.tpu_sc as plsc

D_MODEL = 64
SCALE = math.sqrt(D_MODEL)
VOCAB = 1000000
NC = 2
NS = 16
NW = NC * NS
LANES = 16

VBLK = 128
N_VFULL = VOCAB // VBLK
V_TAIL = VOCAB - N_VFULL * VBLK


def _mesh():
    return plsc.VectorSubcoreMesh(core_axis_name="c", subcore_axis_name="s",
                                  num_cores=NC, num_subcores=NS)


def _iotas():
    return [lax.iota(jnp.int32, LANES) + (16 * c) for c in range(4)]


def _relayout_kernel():
    @functools.partial(
        pl.kernel,
        out_type=jax.ShapeDtypeStruct((VOCAB // 2, 128), jnp.float32),
        mesh=_mesh(),
        scratch_types=[
            pltpu.VMEM((D_MODEL, VBLK), jnp.float32),
            pltpu.VMEM((VBLK // 2, 128), jnp.float32),
        ],
        compiler_params=pltpu.CompilerParams(use_tc_tiling_on_sc=True,
                                             needs_layout_passes=False),
    )
    def relayout(tt_hbm, tail_hbm, out_hbm, inb, outb):
        wid = lax.axis_index("s") * NC + lax.axis_index("c")
        rows_c = _iotas()

        def do_block(blk):
            pltpu.sync_copy(
                tt_hbm.at[slice(None), pl.ds(blk * VBLK, VBLK)], inb)

            @plsc.parallel_loop(0, VBLK // 2, 1, unroll=2)
            def _row(w):
                for c in range(8):
                    col = 2 * w + (1 if c >= 4 else 0)
                    cols = jnp.full((LANES,), col, jnp.int32)
                    v = plsc.load_gather(inb, [rows_c[c % 4], cols])
                    outb[w, pl.ds(16 * c, 16)] = v * SCALE

            pltpu.sync_copy(outb, out_hbm.at[pl.ds(blk * (VBLK // 2),
                                                   VBLK // 2)])

        @pl.loop(0, (N_VFULL + NW - 1) // NW)
        def _blocks(i):
            blk = wid + NW * i

            @pl.when(blk < N_VFULL)
            def _():
                do_block(blk)

        @pl.when(wid == 0)
        def _tail():
            pltpu.sync_copy(tail_hbm, outb.at[pl.ds(0, V_TAIL // 2)])
            pltpu.sync_copy(outb.at[pl.ds(0, V_TAIL // 2)],
                            out_hbm.at[pl.ds(N_VFULL * (VBLK // 2),
                                             V_TAIL // 2)])

    return relayout


def _gather_kernel():
    n_blocks = 200 * 32
    per_w = n_blocks // NW

    @functools.partial(
        pl.kernel,
        out_type=jax.ShapeDtypeStruct((200, 8, 32, 8, 128), jnp.float32),
        mesh=_mesh(),
        scratch_types=[
            pltpu.VMEM((128,), jnp.int32),
            pltpu.VMEM((128, D_MODEL), jnp.float32),
            pltpu.VMEM((8, 8, 128), jnp.float32),
            pltpu.SemaphoreType.DMA,
        ],
        compiler_params=pltpu.CompilerParams(use_tc_tiling_on_sc=False,
                                             needs_layout_passes=False),
    )
    def gather(xt_hbm, tab_hbm, out_hbm, idxb, gb, outb, gsem):
        wid = lax.axis_index("s") * NC + lax.axis_index("c")
        rows_c = _iotas()

        @pl.loop(0, per_w)
        def _blocks(i):
            blk = wid * per_w + i
            s = blk // 32
            bt = blk % 32
            pltpu.sync_copy(xt_hbm.at[s, pl.ds(128 * bt, 128)], idxb)
            pltpu.async_copy(tab_hbm.at[idxb], gb, gsem)
            pltpu.make_async_copy(tab_hbm.at[idxb], gb, gsem).wait()

            @plsc.parallel_loop(0, D_MODEL, 1, unroll=2)
            def _col(col):
                dt = col // 8
                r = col % 8
                cols = jnp.full((LANES,), col, jnp.int32)
                for c in range(8):
                    v = plsc.load_gather(gb, [rows_c[c % 4] + 64 * (c // 4),
                                              cols])
                    outb[dt, r, pl.ds(16 * c, 16)] = v

            pltpu.sync_copy(outb, out_hbm.at[s, slice(None), bt])

    return gather


def kernel(x, table):
    b, s = x.shape
    tail = (lax.slice(table, (N_VFULL * VBLK, 0), (VOCAB, D_MODEL))
            * SCALE).reshape(V_TAIL // 2, 128)
    tab = _relayout_kernel()(table.T, tail)
    out5 = _gather_kernel()(x.T, tab.reshape(VOCAB, D_MODEL))
    return jnp.transpose(out5, (2, 4, 0, 1, 3)).reshape(b, s, D_MODEL)


# speedup vs baseline: 1.3914x; 1.3914x over previous
import functools
import math

import jax
import jax.numpy as jnp
from jax import lax
from jax.experimental import pallas as pl
from jax.experimental.pallas import tpu as pltpu
import jax.experimental.pallas.tpu_sc as plsc

D_MODEL = 64
SCALE = math.sqrt(D_MODEL)
VOCAB = 1000000
NC = 2
NS = 16
NW = NC * NS
LANES = 16

VBLK = 128
N_VFULL = VOCAB // VBLK
V_TAIL = VOCAB - N_VFULL * VBLK
A_PER_W = N_VFULL // NW
A_EXTRA = N_VFULL - A_PER_W * NW


def _mesh():
    return plsc.VectorSubcoreMesh(core_axis_name="c", subcore_axis_name="s",
                                  num_cores=NC, num_subcores=NS)


def _iotas():
    return [lax.iota(jnp.int32, LANES) + (16 * c) for c in range(4)]


def _relayout_kernel():
    @functools.partial(
        pl.kernel,
        out_type=jax.ShapeDtypeStruct((VOCAB // 2, 128), jnp.float32),
        mesh=_mesh(),
        scratch_types=[
            pltpu.VMEM((D_MODEL, VBLK), jnp.float32),
            pltpu.VMEM((D_MODEL, VBLK), jnp.float32),
            pltpu.VMEM((VBLK // 2, 128), jnp.float32),
            pltpu.VMEM((VBLK // 2, 128), jnp.float32),
            pltpu.SemaphoreType.DMA,
            pltpu.SemaphoreType.DMA,
            pltpu.SemaphoreType.DMA,
            pltpu.SemaphoreType.DMA,
        ],
        compiler_params=pltpu.CompilerParams(use_tc_tiling_on_sc=True,
                                             needs_layout_passes=False),
    )
    def relayout(tt_hbm, tail_hbm, out_hbm, inb0, inb1, outb0, outb1,
                 isem0, isem1, osem0, osem1):
        wid = lax.axis_index("s") * NC + lax.axis_index("c")
        rows_c = _iotas()
        inb = (inb0, inb1)
        outb = (outb0, outb1)
        isem = (isem0, isem1)
        osem = (osem0, osem1)

        def start_in(i, b):
            blk = wid + NW * i
            pltpu.async_copy(tt_hbm.at[slice(None), pl.ds(blk * VBLK, VBLK)],
                             inb[b], isem[b])

        def wait_in(b):
            pltpu.make_async_copy(
                tt_hbm.at[slice(None), pl.ds(0, VBLK)], inb[b],
                isem[b]).wait()

        def start_out(i, b):
            blk = wid + NW * i
            pltpu.async_copy(outb[b],
                             out_hbm.at[pl.ds(blk * (VBLK // 2), VBLK // 2)],
                             osem[b])

        def wait_out(b):
            pltpu.make_async_copy(
                outb[b], out_hbm.at[pl.ds(0, VBLK // 2)], osem[b]).wait()

        def transpose(b):
            @plsc.parallel_loop(0, VBLK // 2, 1, unroll=2)
            def _row(w):
                for c in range(8):
                    col = 2 * w + (1 if c >= 4 else 0)
                    cols = jnp.full((LANES,), col, jnp.int32)
                    v = plsc.load_gather(inb[b], [rows_c[c % 4], cols])
                    outb[b][w, pl.ds(16 * c, 16)] = v * SCALE

        n = A_PER_W
        start_in(0, 0)
        start_in(1, 1)
        for i in (0, 1):
            wait_in(i)
            transpose(i)
            start_out(i, i)
            start_in(i + 2, i)

        @pl.loop(2, n - 2, step=2)
        def _group(g):
            for b in range(2):
                i = g + b
                wait_in(b)
                wait_out(b)
                transpose(b)
                start_out(i, b)
                start_in(i + 2, b)

        for i in (n - 2, n - 1):
            b = i % 2
            wait_in(b)
            wait_out(b)
            transpose(b)
            start_out(i, b)
        for b in range(2):
            wait_out(b)

        @pl.when(wid < A_EXTRA)
        def _extra():
            start_in(n, 0)
            wait_in(0)
            transpose(0)
            start_out(n, 0)
            wait_out(0)

        @pl.when(wid == A_EXTRA)
        def _tail():
            pltpu.sync_copy(tail_hbm, outb0.at[pl.ds(0, V_TAIL // 2)])
            pltpu.sync_copy(outb0.at[pl.ds(0, V_TAIL // 2)],
                            out_hbm.at[pl.ds(N_VFULL * (VBLK // 2),
                                             V_TAIL // 2)])

    return relayout


def _gather_kernel():
    per_w = 6400 // NW

    @functools.partial(
        pl.kernel,
        out_type=jax.ShapeDtypeStruct((200, 8, 32, 8, 128), jnp.float32),
        mesh=_mesh(),
        scratch_types=[
            pltpu.VMEM((128,), jnp.int32),
            pltpu.VMEM((128,), jnp.int32),
            pltpu.VMEM((128, D_MODEL), jnp.float32),
            pltpu.VMEM((128, D_MODEL), jnp.float32),
            pltpu.VMEM((D_MODEL, 128), jnp.float32),
            pltpu.VMEM((D_MODEL, 128), jnp.float32),
            pltpu.SemaphoreType.DMA,
            pltpu.SemaphoreType.DMA,
            pltpu.SemaphoreType.DMA,
            pltpu.SemaphoreType.DMA,
        ],
        compiler_params=pltpu.CompilerParams(use_tc_tiling_on_sc=False,
                                             needs_layout_passes=False),
    )
    def gather(xf_hbm, tab_hbm, out_hbm, idx0, idx1, gb0, gb1, ob0, ob1,
               gsem0, gsem1, osem0, osem1):
        wid = lax.axis_index("s") * NC + lax.axis_index("c")
        rows_c = _iotas()
        idxb = (idx0, idx1)
        gb = (gb0, gb1)
        ob = (ob0, ob1)
        gsem = (gsem0, gsem1)
        osem = (osem0, osem1)

        def start_g(i, b):
            blk = wid * per_w + i
            pltpu.sync_copy(xf_hbm.at[blk], idxb[b])
            pltpu.async_copy(tab_hbm.at[idxb[b]], gb[b], gsem[b])

        def wait_g(b):
            pltpu.make_async_copy(tab_hbm.at[idxb[b]], gb[b], gsem[b]).wait()

        def start_out(i, b):
            blk = wid * per_w + i
            s = blk // 32
            bt = blk % 32
            for dt in range(8):
                pltpu.async_copy(ob[b].at[pl.ds(8 * dt, 8)],
                                 out_hbm.at[s, dt, bt], osem[b])

        def wait_out(b):
            for _ in range(8):
                pltpu.make_async_copy(ob[b].at[pl.ds(0, 8)],
                                      out_hbm.at[0, 0, 0], osem[b]).wait()

        def transpose(b):
            @plsc.parallel_loop(0, D_MODEL, 1, unroll=2)
            def _col(col):
                cols = jnp.full((LANES,), col, jnp.int32)
                for c in range(8):
                    v = plsc.load_gather(
                        gb[b], [rows_c[c % 4] + 64 * (c // 4), cols])
                    ob[b][col, pl.ds(16 * c, 16)] = v

        n = per_w
        start_g(0, 0)
        start_g(1, 1)
        for i in (0, 1):
            wait_g(i)
            transpose(i)
            start_out(i, i)
            start_g(i + 2, i)

        @pl.loop(2, n - 2, step=2)
        def _group(g):
            for b in range(2):
                i = g + b
                wait_g(b)
                wait_out(b)
                transpose(b)
                start_out(i, b)
                start_g(i + 2, b)

        for i in (n - 2, n - 1):
            b = i % 2
            wait_g(b)
            wait_out(b)
            transpose(b)
            start_out(i, b)
        for b in range(2):
            wait_out(b)

    return gather


def kernel(x, table):
    b, s = x.shape
    tail = (lax.slice(table, (N_VFULL * VBLK, 0), (VOCAB, D_MODEL))
            * SCALE).reshape(V_TAIL // 2, 128)
    tab = _relayout_kernel()(table.T, tail)
    xf = x.T.reshape((b * s) // 128, 128)
    out5 = _gather_kernel()(xf, tab.reshape(VOCAB, D_MODEL))
    return jnp.transpose(out5, (2, 4, 0, 1, 3)).reshape(b, s, D_MODEL)


# speedup vs baseline: 1.4921x; 1.0724x over previous
import functools
import math

import jax
import jax.numpy as jnp
from jax import lax
from jax.experimental import pallas as pl
from jax.experimental.pallas import tpu as pltpu
import jax.experimental.pallas.tpu_sc as plsc

D_MODEL = 64
SCALE = math.sqrt(D_MODEL)
VOCAB = 1000000
NC = 2
NS = 16
NW = NC * NS
LANES = 16

VBLK = 128
N_VFULL = VOCAB // VBLK
V_TAIL = VOCAB - N_VFULL * VBLK
A_PER_W = N_VFULL // NW
A_EXTRA = N_VFULL - A_PER_W * NW


def _mesh():
    return plsc.VectorSubcoreMesh(core_axis_name="c", subcore_axis_name="s",
                                  num_cores=NC, num_subcores=NS)


def _iotas():
    return [lax.iota(jnp.int32, LANES) + (16 * c) for c in range(4)]


def _relayout_kernel():
    @functools.partial(
        pl.kernel,
        out_type=jax.ShapeDtypeStruct((VOCAB // 2, 128), jnp.float32),
        mesh=_mesh(),
        scratch_types=[
            pltpu.VMEM((D_MODEL, VBLK), jnp.float32),
            pltpu.VMEM((D_MODEL, VBLK), jnp.float32),
            pltpu.VMEM((VBLK // 2, 128), jnp.float32),
            pltpu.VMEM((VBLK // 2, 128), jnp.float32),
            pltpu.SemaphoreType.DMA,
            pltpu.SemaphoreType.DMA,
            pltpu.SemaphoreType.DMA,
            pltpu.SemaphoreType.DMA,
        ],
        compiler_params=pltpu.CompilerParams(use_tc_tiling_on_sc=True,
                                             needs_layout_passes=False),
    )
    def relayout(tt_hbm, tail_hbm, out_hbm, inb0, inb1, outb0, outb1,
                 isem0, isem1, osem0, osem1):
        wid = lax.axis_index("s") * NC + lax.axis_index("c")
        rows_c = _iotas()
        inb = (inb0, inb1)
        outb = (outb0, outb1)
        isem = (isem0, isem1)
        osem = (osem0, osem1)

        def start_in(i, b):
            blk = wid + NW * i
            pltpu.async_copy(tt_hbm.at[slice(None), pl.ds(blk * VBLK, VBLK)],
                             inb[b], isem[b])

        def wait_in(b):
            pltpu.make_async_copy(
                tt_hbm.at[slice(None), pl.ds(0, VBLK)], inb[b],
                isem[b]).wait()

        def start_out(i, b):
            blk = wid + NW * i
            pltpu.async_copy(outb[b],
                             out_hbm.at[pl.ds(blk * (VBLK // 2), VBLK // 2)],
                             osem[b])

        def wait_out(b):
            pltpu.make_async_copy(
                outb[b], out_hbm.at[pl.ds(0, VBLK // 2)], osem[b]).wait()

        def transpose(b):
            @plsc.parallel_loop(0, VBLK // 2, 1, unroll=4)
            def _row(w):
                for c in range(8):
                    col = 2 * w + (1 if c >= 4 else 0)
                    cols = jnp.full((LANES,), col, jnp.int32)
                    v = plsc.load_gather(inb[b], [rows_c[c % 4], cols])
                    outb[b][w, pl.ds(16 * c, 16)] = v * SCALE

        n = A_PER_W
        start_in(0, 0)
        start_in(1, 1)
        for i in (0, 1):
            wait_in(i)
            transpose(i)
            start_out(i, i)
            start_in(i + 2, i)

        @pl.loop(2, n - 2, step=2)
        def _group(g):
            for b in range(2):
                i = g + b
                wait_in(b)
                wait_out(b)
                transpose(b)
                start_out(i, b)
                start_in(i + 2, b)

        for i in (n - 2, n - 1):
            b = i % 2
            wait_in(b)
            wait_out(b)
            transpose(b)
            start_out(i, b)
        for b in range(2):
            wait_out(b)

        @pl.when(wid < A_EXTRA)
        def _extra():
            start_in(n, 0)
            wait_in(0)
            transpose(0)
            start_out(n, 0)
            wait_out(0)

        @pl.when(wid == A_EXTRA)
        def _tail():
            pltpu.sync_copy(tail_hbm, outb0.at[pl.ds(0, V_TAIL // 2)])
            pltpu.sync_copy(outb0.at[pl.ds(0, V_TAIL // 2)],
                            out_hbm.at[pl.ds(N_VFULL * (VBLK // 2),
                                             V_TAIL // 2)])

    return relayout


def _gather_kernel():
    per_w = 6400 // NW

    @functools.partial(
        pl.kernel,
        out_type=jax.ShapeDtypeStruct((200, 8, 32, 8, 128), jnp.float32),
        mesh=_mesh(),
        scratch_types=[
            pltpu.VMEM((200, 128), jnp.int32),
            pltpu.VMEM((128, D_MODEL), jnp.float32),
            pltpu.VMEM((128, D_MODEL), jnp.float32),
            pltpu.VMEM((D_MODEL, 128), jnp.float32),
            pltpu.VMEM((D_MODEL, 128), jnp.float32),
            pltpu.SemaphoreType.DMA,
            pltpu.SemaphoreType.DMA,
            pltpu.SemaphoreType.DMA,
            pltpu.SemaphoreType.DMA,
            pltpu.SemaphoreType.DMA,
        ],
        compiler_params=pltpu.CompilerParams(use_tc_tiling_on_sc=False,
                                             needs_layout_passes=False),
    )
    def gather(xf_hbm, tab_hbm, out_hbm, idxa, gb0, gb1, ob0, ob1,
               isem, gsem0, gsem1, osem0, osem1):
        wid = lax.axis_index("s") * NC + lax.axis_index("c")
        rows_c = _iotas()
        gb = (gb0, gb1)
        ob = (ob0, ob1)
        gsem = (gsem0, gsem1)
        osem = (osem0, osem1)

        pltpu.async_copy(xf_hbm.at[pl.ds(wid * per_w, per_w)], idxa, isem)
        pltpu.make_async_copy(xf_hbm.at[pl.ds(0, per_w)], idxa, isem).wait()

        def start_g(i, b):
            pltpu.async_copy(tab_hbm.at[idxa.at[i]], gb[b], gsem[b])

        def wait_g(b):
            pltpu.make_async_copy(tab_hbm.at[idxa.at[0]], gb[b],
                                  gsem[b]).wait()

        def start_out(i, b):
            blk = wid * per_w + i
            s = blk // 32
            bt = blk % 32
            for dt in range(8):
                pltpu.async_copy(ob[b].at[pl.ds(8 * dt, 8)],
                                 out_hbm.at[s, dt, bt], osem[b])

        def wait_out(b):
            for _ in range(8):
                pltpu.make_async_copy(ob[b].at[pl.ds(0, 8)],
                                      out_hbm.at[0, 0, 0], osem[b]).wait()

        def transpose(b):
            @plsc.parallel_loop(0, D_MODEL, 1, unroll=4)
            def _col(col):
                cols = jnp.full((LANES,), col, jnp.int32)
                for c in range(8):
                    v = plsc.load_gather(
                        gb[b], [rows_c[c % 4] + 64 * (c // 4), cols])
                    ob[b][col, pl.ds(16 * c, 16)] = v

        n = per_w
        start_g(0, 0)
        start_g(1, 1)
        for i in (0, 1):
            wait_g(i)
            transpose(i)
            start_out(i, i)
            start_g(i + 2, i)

        @pl.loop(2, n - 2, step=2)
        def _group(g):
            for b in range(2):
                i = g + b
                wait_g(b)
                wait_out(b)
                transpose(b)
                start_out(i, b)
                start_g(i + 2, b)

        for i in (n - 2, n - 1):
            b = i % 2
            wait_g(b)
            wait_out(b)
            transpose(b)
            start_out(i, b)
        for b in range(2):
            wait_out(b)

    return gather


def kernel(x, table):
    b, s = x.shape
    tail = (lax.slice(table, (N_VFULL * VBLK, 0), (VOCAB, D_MODEL))
            * SCALE).reshape(V_TAIL // 2, 128)
    tab = _relayout_kernel()(table.T, tail)
    xf = x.T.reshape((b * s) // 128, 128)
    out5 = _gather_kernel()(xf, tab.reshape(VOCAB, D_MODEL))
    return jnp.transpose(out5, (2, 4, 0, 1, 3)).reshape(b, s, D_MODEL)


# speedup vs baseline: 1.7254x; 1.1564x over previous
import functools
import math

import jax
import jax.numpy as jnp
from jax import lax
from jax.experimental import pallas as pl
from jax.experimental.pallas import tpu as pltpu
import jax.experimental.pallas.tpu_sc as plsc

D_MODEL = 64
SCALE = math.sqrt(D_MODEL)
VOCAB = 1000000
NC = 2
NS = 16
NW = NC * NS
LANES = 16

VBLK = 128
N_VFULL = VOCAB // VBLK
V_TAIL = VOCAB - N_VFULL * VBLK
A_PER_W = N_VFULL // NW
A_EXTRA = N_VFULL - A_PER_W * NW


def _mesh():
    return plsc.VectorSubcoreMesh(core_axis_name="c", subcore_axis_name="s",
                                  num_cores=NC, num_subcores=NS)


def _iotas():
    return [lax.iota(jnp.int32, LANES) + (16 * c) for c in range(4)]


def _relayout_kernel():
    @functools.partial(
        pl.kernel,
        out_type=jax.ShapeDtypeStruct((VOCAB // 2, 128), jnp.float32),
        mesh=_mesh(),
        scratch_types=[
            pltpu.VMEM((D_MODEL, VBLK), jnp.float32),
            pltpu.VMEM((D_MODEL, VBLK), jnp.float32),
            pltpu.VMEM((VBLK // 2, 128), jnp.float32),
            pltpu.VMEM((VBLK // 2, 128), jnp.float32),
            pltpu.SemaphoreType.DMA,
            pltpu.SemaphoreType.DMA,
            pltpu.SemaphoreType.DMA,
            pltpu.SemaphoreType.DMA,
        ],
        compiler_params=pltpu.CompilerParams(use_tc_tiling_on_sc=True,
                                             needs_layout_passes=False),
    )
    def relayout(tt_hbm, tail_hbm, out_hbm, inb0, inb1, outb0, outb1,
                 isem0, isem1, osem0, osem1):
        wid = lax.axis_index("s") * NC + lax.axis_index("c")
        rows_c = _iotas()
        inb = (inb0, inb1)
        outb = (outb0, outb1)
        isem = (isem0, isem1)
        osem = (osem0, osem1)

        def start_in(i, b):
            blk = wid + NW * i
            pltpu.async_copy(tt_hbm.at[slice(None), pl.ds(blk * VBLK, VBLK)],
                             inb[b], isem[b])

        def wait_in(b):
            pltpu.make_async_copy(
                tt_hbm.at[slice(None), pl.ds(0, VBLK)], inb[b],
                isem[b]).wait()

        def start_out(i, b):
            blk = wid + NW * i
            pltpu.async_copy(outb[b],
                             out_hbm.at[pl.ds(blk * (VBLK // 2), VBLK // 2)],
                             osem[b])

        def wait_out(b):
            pltpu.make_async_copy(
                outb[b], out_hbm.at[pl.ds(0, VBLK // 2)], osem[b]).wait()

        def transpose(b):
            @plsc.parallel_loop(0, VBLK // 2, 1, unroll=4)
            def _row(w):
                for c in range(8):
                    col = 2 * w + (1 if c >= 4 else 0)
                    v = inb[b][pl.ds(16 * (c % 4), 16), pl.ds(col, 1)]
                    outb[b][w, pl.ds(16 * c, 16)] = v.reshape(LANES) * SCALE

        n = A_PER_W
        start_in(0, 0)
        start_in(1, 1)
        for i in (0, 1):
            wait_in(i)
            transpose(i)
            start_out(i, i)
            start_in(i + 2, i)

        @pl.loop(2, n - 2, step=2)
        def _group(g):
            for b in range(2):
                i = g + b
                wait_in(b)
                wait_out(b)
                transpose(b)
                start_out(i, b)
                start_in(i + 2, b)

        for i in (n - 2, n - 1):
            b = i % 2
            wait_in(b)
            wait_out(b)
            transpose(b)
            start_out(i, b)
        for b in range(2):
            wait_out(b)

        @pl.when(wid < A_EXTRA)
        def _extra():
            start_in(n, 0)
            wait_in(0)
            transpose(0)
            start_out(n, 0)
            wait_out(0)

        @pl.when(wid == A_EXTRA)
        def _tail():
            pltpu.sync_copy(tail_hbm, outb0.at[pl.ds(0, V_TAIL // 2)])
            pltpu.sync_copy(outb0.at[pl.ds(0, V_TAIL // 2)],
                            out_hbm.at[pl.ds(N_VFULL * (VBLK // 2),
                                             V_TAIL // 2)])

    return relayout


def _gather_kernel():
    per_w = 6400 // NW

    @functools.partial(
        pl.kernel,
        out_type=jax.ShapeDtypeStruct((200, 8, 32, 8, 128), jnp.float32),
        mesh=_mesh(),
        scratch_types=[
            pltpu.VMEM((200, 128), jnp.int32),
            pltpu.VMEM((128, D_MODEL), jnp.float32),
            pltpu.VMEM((128, D_MODEL), jnp.float32),
            pltpu.VMEM((D_MODEL, 128), jnp.float32),
            pltpu.VMEM((D_MODEL, 128), jnp.float32),
            pltpu.SemaphoreType.DMA,
            pltpu.SemaphoreType.DMA,
            pltpu.SemaphoreType.DMA,
            pltpu.SemaphoreType.DMA,
            pltpu.SemaphoreType.DMA,
        ],
        compiler_params=pltpu.CompilerParams(use_tc_tiling_on_sc=False,
                                             needs_layout_passes=False),
    )
    def gather(xf_hbm, tab_hbm, out_hbm, idxa, gb0, gb1, ob0, ob1,
               isem, gsem0, gsem1, osem0, osem1):
        wid = lax.axis_index("s") * NC + lax.axis_index("c")
        rows_c = _iotas()
        gb = (gb0, gb1)
        ob = (ob0, ob1)
        gsem = (gsem0, gsem1)
        osem = (osem0, osem1)

        pltpu.async_copy(xf_hbm.at[pl.ds(wid * per_w, per_w)], idxa, isem)
        pltpu.make_async_copy(xf_hbm.at[pl.ds(0, per_w)], idxa, isem).wait()

        def start_g(i, b):
            pltpu.async_copy(tab_hbm.at[idxa.at[i]], gb[b], gsem[b])

        def wait_g(b):
            pltpu.make_async_copy(tab_hbm.at[idxa.at[0]], gb[b],
                                  gsem[b]).wait()

        def start_out(i, b):
            blk = wid * per_w + i
            s = blk // 32
            bt = blk % 32
            for dt in range(8):
                pltpu.async_copy(ob[b].at[pl.ds(8 * dt, 8)],
                                 out_hbm.at[s, dt, bt], osem[b])

        def wait_out(b):
            for _ in range(8):
                pltpu.make_async_copy(ob[b].at[pl.ds(0, 8)],
                                      out_hbm.at[0, 0, 0], osem[b]).wait()

        def transpose(b):
            @plsc.parallel_loop(0, D_MODEL, 1, unroll=4)
            def _col(col):
                cols = jnp.full((LANES,), col, jnp.int32)
                for c in range(8):
                    v = plsc.load_gather(
                        gb[b], [rows_c[c % 4] + 64 * (c // 4), cols])
                    ob[b][col, pl.ds(16 * c, 16)] = v * SCALE

        n = per_w
        start_g(0, 0)
        start_g(1, 1)
        for i in (0, 1):
            wait_g(i)
            transpose(i)
            start_out(i, i)
            start_g(i + 2, i)

        @pl.loop(2, n - 2, step=2)
        def _group(g):
            for b in range(2):
                i = g + b
                wait_g(b)
                wait_out(b)
                transpose(b)
                start_out(i, b)
                start_g(i + 2, b)

        for i in (n - 2, n - 1):
            b = i % 2
            wait_g(b)
            wait_out(b)
            transpose(b)
            start_out(i, b)
        for b in range(2):
            wait_out(b)

    return gather


def kernel(x, table):
    b, s = x.shape
    xf = x.T.reshape((b * s) // 128, 128)
    out5 = _gather_kernel()(xf, table)
    return jnp.transpose(out5, (2, 4, 0, 1, 3)).reshape(b, s, D_MODEL)
